# baseline (speedup 1.0000x reference)
import jax
import jax.numpy as jnp
from jax.experimental import pallas as pl
from jax.experimental.pallas import tpu as pltpu

_N = 8192
_B = 32
_K = 128
_M = 128
_TOPK = 5
_NCHUNKS = 4
_C = _N // _NCHUNKS

_NT = (((1,), (1,)), ((), ()))
_LOG_EPS = -36.8413614879047


def _gcl_kernel(kb_ref, k_ref, beta_ref, gamma_ref, a_ref, content_ref,
                out_ref, scratch_ref):
    i = pl.program_id(0)

    @pl.when(i < _NCHUNKS)
    def _phase_a():
        kb = kb_ref[:, :]
        k = k_ref[:, :]
        beta = beta_ref[:, :]
        scores = jax.lax.dot_general(k, kb, _NT,
                                     preferred_element_type=jnp.float32)
        ones = jnp.ones((1, _K), dtype=jnp.float32)
        rn2 = jax.lax.dot_general(ones, kb * kb, _NT,
                                  preferred_element_type=jnp.float32)
        rk = jnp.sqrt(jnp.sum(k * k, axis=1, keepdims=True))
        denom = jnp.maximum(jnp.sqrt(rn2) * rk, 1e-8)
        scratch_ref[:, pl.ds(i * _C, _C)] = beta * (scores / denom)

    @pl.when(i == _NCHUNKS)
    def _weights():
        logits = scratch_ref[:, :]
        e = jnp.exp(logits)
        cur = logits
        t5 = None
        for _ in range(_TOPK):
            t5 = jnp.max(cur, axis=1, keepdims=True)
            cur = jnp.where(cur == t5, -jnp.inf, cur)
        sel = logits >= t5
        em = e * jnp.where(sel, 1.0, 1e-16)
        s1 = jnp.sum(em, axis=1, keepdims=True)
        gamma = gamma_ref[:, :]
        logf = jnp.where(sel, 0.0, _LOG_EPS)
        w = jnp.exp(gamma * ((logits + logf) - jnp.log(s1)))
        w = w / jnp.sum(w, axis=1, keepdims=True)
        w2 = w * w
        sw2 = jnp.sum(w2, axis=1, keepdims=True)
        scratch_ref[:, :] = w - w2
        out_ref[:, :] = sw2 * a_ref[:, :]

    @pl.when(i >= _NCHUNKS)
    def _phase_b():
        c = i - _NCHUNKS
        v = scratch_ref[:, pl.ds(c * _C, _C)]
        out_ref[:, :] += jnp.dot(v, content_ref[:, :],
                                 preferred_element_type=jnp.float32)


def kernel(k, beta, g, s, gamma, a, a_k, content_bias, key_bias, candidates):
    del g, s, a_k, candidates
    nc = _NCHUNKS
    return pl.pallas_call(
        _gcl_kernel,
        grid=(2 * nc,),
        in_specs=[
            pl.BlockSpec((_C, _K), lambda i: (jnp.minimum(i, nc - 1), 0)),
            pl.BlockSpec((_B, _K), lambda i: (0, 0)),
            pl.BlockSpec((_B, 1), lambda i: (0, 0)),
            pl.BlockSpec((_B, 1), lambda i: (0, 0)),
            pl.BlockSpec((_B, _M), lambda i: (0, 0)),
            pl.BlockSpec((_C, _M), lambda i: (jnp.maximum(i - nc, 0), 0)),
        ],
        out_specs=pl.BlockSpec((_B, _M), lambda i: (0, 0)),
        out_shape=jax.ShapeDtypeStruct((_B, _M), jnp.float32),
        scratch_shapes=[pltpu.VMEM((_B, _N), jnp.float32)],
        compiler_params=pltpu.CompilerParams(
            dimension_semantics=("arbitrary",)),
    )(key_bias, k, beta, gamma, a, content_bias)


# speedup vs baseline: 1.0302x; 1.0302x over previous
import jax
import jax.numpy as jnp
from jax.experimental import pallas as pl
from jax.experimental.pallas import tpu as pltpu

_N = 8192
_B = 32
_K = 128
_M = 128
_TOPK = 5
_NCHUNKS = 4
_C = _N // _NCHUNKS

_NT = (((1,), (1,)), ((), ()))
_LOG_EPS = -36.8413614879047


def _gcl_kernel(kb_ref, k_ref, beta_ref, gamma_ref, a_ref, content_ref,
                out_ref, scratch_ref):
    i = pl.program_id(0)

    @pl.when(i < _NCHUNKS)
    def _phase_a():
        kb = kb_ref[:, :]
        k = k_ref[:, :]
        beta = beta_ref[:, :]
        scores = jax.lax.dot_general(k, kb, _NT,
                                     preferred_element_type=jnp.float32)
        ones = jnp.ones((1, _K), dtype=jnp.float32)
        rn2 = jax.lax.dot_general(ones, kb * kb, _NT,
                                  preferred_element_type=jnp.float32)
        rk = jnp.sqrt(jnp.sum(k * k, axis=1, keepdims=True))
        denom = jnp.maximum(jnp.sqrt(rn2) * rk, 1e-8)
        scratch_ref[:, pl.ds(i * _C, _C)] = beta * (scores / denom)

    @pl.when(i == _NCHUNKS)
    def _weights():
        logits = scratch_ref[:, :]
        cur = logits
        t5 = None
        for _ in range(_TOPK):
            t5 = jnp.max(cur, axis=1, keepdims=True)
            cur = jnp.where(cur == t5, -jnp.inf, cur)
        gamma = gamma_ref[:, :]
        logf = jnp.where(logits >= t5, 0.0, _LOG_EPS)
        u = jnp.exp(gamma * (logits + logf))
        u2 = u * u
        winv = 1.0 / jnp.sum(u, axis=1, keepdims=True)
        sw2 = jnp.sum(u2, axis=1, keepdims=True) * (winv * winv)
        scratch_ref[:, :] = winv * u - (winv * winv) * u2
        out_ref[:, :] = sw2 * a_ref[:, :]

    @pl.when(i >= _NCHUNKS)
    def _phase_b():
        c = i - _NCHUNKS
        v = scratch_ref[:, pl.ds(c * _C, _C)]
        out_ref[:, :] += jnp.dot(v, content_ref[:, :],
                                 preferred_element_type=jnp.float32)


def kernel(k, beta, g, s, gamma, a, a_k, content_bias, key_bias, candidates):
    del g, s, a_k, candidates
    nc = _NCHUNKS
    return pl.pallas_call(
        _gcl_kernel,
        grid=(2 * nc,),
        in_specs=[
            pl.BlockSpec((_C, _K), lambda i: (jnp.minimum(i, nc - 1), 0)),
            pl.BlockSpec((_B, _K), lambda i: (0, 0)),
            pl.BlockSpec((_B, 1), lambda i: (0, 0)),
            pl.BlockSpec((_B, 1), lambda i: (0, 0)),
            pl.BlockSpec((_B, _M), lambda i: (0, 0)),
            pl.BlockSpec((_C, _M), lambda i: (jnp.maximum(i - nc, 0), 0)),
        ],
        out_specs=pl.BlockSpec((_B, _M), lambda i: (0, 0)),
        out_shape=jax.ShapeDtypeStruct((_B, _M), jnp.float32),
        scratch_shapes=[pltpu.VMEM((_B, _N), jnp.float32)],
        compiler_params=pltpu.CompilerParams(
            dimension_semantics=("arbitrary",)),
    )(key_bias, k, beta, gamma, a, content_bias)


# speedup vs baseline: 1.1810x; 1.1464x over previous
import jax
import jax.numpy as jnp
from jax.experimental import pallas as pl
from jax.experimental.pallas import tpu as pltpu

_N = 8192
_B = 32
_K = 128
_M = 128
_TOPK = 5
_H = _N // 2
_C = _H // 2
_NT = (((1,), (1,)), ((), ()))
_LOG_EPS = -36.8413614879047


def _gcl_kernel(kb1_ref, kb2_ref, k_ref, beta_ref, gamma_ref, a_ref,
                ct1_ref, ct2_ref, out_ref, scratch_ref):
    i = pl.program_id(0)

    def _logits(kb, k, beta, rk):
        scores = jax.lax.dot_general(k, kb, _NT,
                                     preferred_element_type=jnp.float32)
        ones = jnp.ones((1, _K), dtype=jnp.float32)
        rn2 = jax.lax.dot_general(ones, kb * kb, _NT,
                                  preferred_element_type=jnp.float32)
        denom = jnp.maximum(jnp.sqrt(rn2) * rk, 1e-8)
        return beta * (scores / denom)

    @pl.when(i < 2)
    def _phase_a():
        k = k_ref[:, :]
        beta = beta_ref[:, :]
        rk = jnp.sqrt(jnp.sum(k * k, axis=1, keepdims=True))
        scratch_ref[:, pl.ds(i * _C, _C)] = _logits(kb1_ref[:, :], k, beta, rk)
        scratch_ref[:, pl.ds(_H + i * _C, _C)] = _logits(kb2_ref[:, :], k, beta, rk)

    @pl.when(i == 2)
    def _weights():
        logits = scratch_ref[:, :]
        cur = logits
        t5 = None
        for _ in range(_TOPK):
            t5 = jnp.max(cur, axis=1, keepdims=True)
            cur = jnp.where(cur == t5, -jnp.inf, cur)
        gamma = gamma_ref[:, :]
        logf = jnp.where(logits >= t5, 0.0, _LOG_EPS)
        u = jnp.exp(gamma * (logits + logf))
        u2 = u * u
        winv = 1.0 / jnp.sum(u, axis=1, keepdims=True)
        sw2 = jnp.sum(u2, axis=1, keepdims=True) * (winv * winv)
        scratch_ref[:, :] = winv * u - (winv * winv) * u2
        out_ref[:, :] = sw2 * a_ref[:, :]

    @pl.when(i >= 2)
    def _phase_b():
        c = i - 2
        v1 = scratch_ref[:, pl.ds(c * _C, _C)]
        v2 = scratch_ref[:, pl.ds(_H + c * _C, _C)]
        out_ref[:, :] += (
            jnp.dot(v1, ct1_ref[:, :], preferred_element_type=jnp.float32)
            + jnp.dot(v2, ct2_ref[:, :], preferred_element_type=jnp.float32))


def kernel(k, beta, g, s, gamma, a, a_k, content_bias, key_bias, candidates):
    del g, s, a_k, candidates
    return pl.pallas_call(
        _gcl_kernel,
        grid=(4,),
        in_specs=[
            pl.BlockSpec((_C, _K), lambda i: (jnp.minimum(i, 1), 0)),
            pl.BlockSpec((_C, _K), lambda i: (2 + jnp.minimum(i, 1), 0)),
            pl.BlockSpec((_B, _K), lambda i: (0, 0)),
            pl.BlockSpec((_B, 1), lambda i: (0, 0)),
            pl.BlockSpec((_B, 1), lambda i: (0, 0)),
            pl.BlockSpec((_B, _M), lambda i: (0, 0)),
            pl.BlockSpec((_C, _M), lambda i: (jnp.maximum(i - 2, 0), 0)),
            pl.BlockSpec((_C, _M), lambda i: (2 + jnp.maximum(i - 2, 0), 0)),
        ],
        out_specs=pl.BlockSpec((_B, _M), lambda i: (0, 0)),
        out_shape=jax.ShapeDtypeStruct((_B, _M), jnp.float32),
        scratch_shapes=[pltpu.VMEM((_B, _N), jnp.float32)],
        compiler_params=pltpu.CompilerParams(
            dimension_semantics=("arbitrary",)),
    )(key_bias, key_bias, k, beta, gamma, a, content_bias, content_bias)


# speedup vs baseline: 1.1909x; 1.0085x over previous
import jax
import jax.numpy as jnp
from jax.experimental import pallas as pl
from jax.experimental.pallas import tpu as pltpu

_N = 8192
_B = 32
_K = 128
_M = 128
_TOPK = 5
_H = _N // 2
_C = _H // 2
_NT = (((1,), (1,)), ((), ()))
_LOG_EPS = -36.8413614879047


def _gcl_kernel(kb1_ref, kb2_ref, k_ref, beta_ref, gamma_ref, a_ref,
                ct1_ref, ct2_ref, out_ref, scratch_ref):
    i = pl.program_id(0)

    def _logits(kb, k, beta, rk):
        scores = jax.lax.dot_general(k, kb, _NT,
                                     preferred_element_type=jnp.float32)
        ones = jnp.ones((1, _K), dtype=jnp.float32)
        rn2 = jax.lax.dot_general(ones, kb * kb, _NT,
                                  preferred_element_type=jnp.float32)
        denom = jnp.maximum(jnp.sqrt(rn2) * rk, 1e-8)
        return beta * (scores / denom)

    @pl.when(i < 2)
    def _phase_a():
        k = k_ref[:, :]
        beta = beta_ref[:, :]
        rk = jnp.sqrt(jnp.sum(k * k, axis=1, keepdims=True))
        scratch_ref[:, pl.ds(i * _C, _C)] = _logits(kb1_ref[:, :], k, beta, rk)
        scratch_ref[:, pl.ds(_H + i * _C, _C)] = _logits(kb2_ref[:, :], k, beta, rk)

    @pl.when(i == 1)
    def _weights():
        logits = scratch_ref[:, :]
        cur = logits
        t5 = None
        for _ in range(_TOPK):
            t5 = jnp.max(cur, axis=1, keepdims=True)
            cur = jnp.where(cur == t5, -jnp.inf, cur)
        gamma = gamma_ref[:, :]
        logf = jnp.where(logits >= t5, 0.0, _LOG_EPS)
        u = jnp.exp(gamma * (logits + logf))
        u2 = u * u
        winv = 1.0 / jnp.sum(u, axis=1, keepdims=True)
        sw2 = jnp.sum(u2, axis=1, keepdims=True) * (winv * winv)
        scratch_ref[:, :] = winv * u - (winv * winv) * u2
        out_ref[:, :] = sw2 * a_ref[:, :]

    @pl.when(i >= 1)
    def _phase_b():
        c = i - 1
        v1 = scratch_ref[:, pl.ds(c * _C, _C)]
        v2 = scratch_ref[:, pl.ds(_H + c * _C, _C)]
        out_ref[:, :] += (
            jnp.dot(v1, ct1_ref[:, :], preferred_element_type=jnp.float32)
            + jnp.dot(v2, ct2_ref[:, :], preferred_element_type=jnp.float32))


def kernel(k, beta, g, s, gamma, a, a_k, content_bias, key_bias, candidates):
    del g, s, a_k, candidates
    return pl.pallas_call(
        _gcl_kernel,
        grid=(3,),
        in_specs=[
            pl.BlockSpec((_C, _K), lambda i: (jnp.minimum(i, 1), 0)),
            pl.BlockSpec((_C, _K), lambda i: (2 + jnp.minimum(i, 1), 0)),
            pl.BlockSpec((_B, _K), lambda i: (0, 0)),
            pl.BlockSpec((_B, 1), lambda i: (0, 0)),
            pl.BlockSpec((_B, 1), lambda i: (0, 0)),
            pl.BlockSpec((_B, _M), lambda i: (0, 0)),
            pl.BlockSpec((_C, _M), lambda i: (jnp.maximum(i - 1, 0), 0)),
            pl.BlockSpec((_C, _M), lambda i: (2 + jnp.maximum(i - 1, 0), 0)),
        ],
        out_specs=pl.BlockSpec((_B, _M), lambda i: (0, 0)),
        out_shape=jax.ShapeDtypeStruct((_B, _M), jnp.float32),
        scratch_shapes=[pltpu.VMEM((_B, _N), jnp.float32)],
        compiler_params=pltpu.CompilerParams(
            dimension_semantics=("arbitrary",)),
    )(key_bias, key_bias, k, beta, gamma, a, content_bias, content_bias)


# speedup vs baseline: 1.2551x; 1.0539x over previous
import jax
import jax.numpy as jnp
from jax.experimental import pallas as pl
from jax.experimental.pallas import tpu as pltpu

_N = 8192
_B = 32
_K = 128
_M = 128
_TOPK = 5
_C = _N // 2
_NT = (((1,), (1,)), ((), ()))
_LOG_EPS = -36.8413614879047


def _gcl_kernel(kb_ref, k_ref, beta_ref, gamma_ref, a_ref,
                ct_ref, out_ref, scratch_ref):
    i = pl.program_id(0)

    @pl.when(i < 2)
    def _phase_a():
        k = k_ref[:, :]
        beta = beta_ref[:, :]
        rk = jnp.sqrt(jnp.sum(k * k, axis=1, keepdims=True))
        kb = kb_ref[:, :]
        scores = jax.lax.dot_general(k, kb, _NT,
                                     preferred_element_type=jnp.float32)
        ones = jnp.ones((1, _K), dtype=jnp.float32)
        rn2 = jax.lax.dot_general(ones, kb * kb, _NT,
                                  preferred_element_type=jnp.float32)
        denom = jnp.maximum(jnp.sqrt(rn2) * rk, 1e-8)
        scratch_ref[:, pl.ds(i * _C, _C)] = beta * (scores / denom)

    @pl.when(i == 1)
    def _weights():
        logits = scratch_ref[:, :]
        cur = logits
        t5 = None
        for _ in range(_TOPK):
            t5 = jnp.max(cur, axis=1, keepdims=True)
            cur = jnp.where(cur == t5, -jnp.inf, cur)
        gamma = gamma_ref[:, :]
        logf = jnp.where(logits >= t5, 0.0, _LOG_EPS)
        u = jnp.exp(gamma * (logits + logf))
        u2 = u * u
        winv = 1.0 / jnp.sum(u, axis=1, keepdims=True)
        sw2 = jnp.sum(u2, axis=1, keepdims=True) * (winv * winv)
        scratch_ref[:, :] = winv * u - (winv * winv) * u2
        out_ref[:, :] = sw2 * a_ref[:, :]

    @pl.when(i >= 1)
    def _phase_b():
        c = i - 1
        v = scratch_ref[:, pl.ds(c * _C, _C)]
        out_ref[:, :] += jnp.dot(v, ct_ref[:, :],
                                 preferred_element_type=jnp.float32)


def kernel(k, beta, g, s, gamma, a, a_k, content_bias, key_bias, candidates):
    del g, s, a_k, candidates
    return pl.pallas_call(
        _gcl_kernel,
        grid=(3,),
        in_specs=[
            pl.BlockSpec((_C, _K), lambda i: (jnp.minimum(i, 1), 0)),
            pl.BlockSpec((_B, _K), lambda i: (0, 0)),
            pl.BlockSpec((_B, 1), lambda i: (0, 0)),
            pl.BlockSpec((_B, 1), lambda i: (0, 0)),
            pl.BlockSpec((_B, _M), lambda i: (0, 0)),
            pl.BlockSpec((_C, _M), lambda i: (jnp.maximum(i - 1, 0), 0)),
        ],
        out_specs=pl.BlockSpec((_B, _M), lambda i: (0, 0)),
        out_shape=jax.ShapeDtypeStruct((_B, _M), jnp.float32),
        scratch_shapes=[pltpu.VMEM((_B, _N), jnp.float32)],
        compiler_params=pltpu.CompilerParams(
            dimension_semantics=("arbitrary",)),
    )(key_bias, k, beta, gamma, a, content_bias)
